# Initial kernel scaffold; baseline (speedup 1.0000x reference)
import jax
import jax.numpy as jnp
from jax.experimental import pallas as pl


def kernel(x, dct_weight, W1, W2, k_tensor):
    raise NotImplementedError("write your pallas kernel here")


# speedup vs baseline: 1.7343x; 1.7343x over previous
import jax
import jax.numpy as jnp
from jax.experimental import pallas as pl

def _pool_body(x_ref, d_ref, s_ref):
    s_ref[...] = jnp.sum(x_ref[...] * d_ref[...][None], axis=-1)


def _gate_body(s_ref, w1_ref, w2_ref, k_ref, b_ref, raw_ref, m_ref, g_ref):
    s = s_ref[...]
    h = jnp.maximum(jnp.dot(s, w1_ref[...].T, preferred_element_type=jnp.float32), 0.0)
    raw = jnp.dot(h, w2_ref[...].T, preferred_element_type=jnp.float32)
    raw_ref[...] = raw
    bounded = jnp.tanh(raw)
    b_ref[...] = bounded

    c = raw.shape[1]
    bits = jax.lax.bitcast_convert_type(raw + 0.0, jnp.int32)
    key = bits ^ ((bits >> 31) & jnp.int32(0x7FFFFFFF))
    k = k_ref[...]

    cnt0 = jnp.sum((key >= 0).astype(jnp.int32), axis=1, keepdims=True)
    base = jnp.where(cnt0 >= k, jnp.int32(0), jnp.int32(-2147483648))
    for bit in range(30, -1, -1):
        cand = base + jnp.int32(1 << bit)
        cnt = jnp.sum((key >= cand).astype(jnp.int32), axis=1, keepdims=True)
        base = jnp.where(cnt >= k, cand, base)

    gt = key > base
    eq = key == base
    cnt_gt = jnp.sum(gt.astype(jnp.int32), axis=1, keepdims=True)
    rows = jax.lax.broadcasted_iota(jnp.int32, (c, c), 0)
    cols = jax.lax.broadcasted_iota(jnp.int32, (c, c), 1)
    tri = (rows < cols).astype(jnp.float32)
    eq_prefix = jnp.dot(eq.astype(jnp.float32), tri,
                        preferred_element_type=jnp.float32)
    n_eq = (k - cnt_gt).astype(jnp.float32)
    mask = jnp.where(gt | (eq & (eq_prefix < n_eq)), 1.0, 0.0)
    m_ref[...] = mask
    g_ref[...] = bounded + (mask - bounded)


def _scale_body(x_ref, g_ref, o_ref):
    o_ref[...] = x_ref[...] * g_ref[...][:, :, None]


def kernel(x, dct_weight, W1, W2, k_tensor):
    b, c, dh, dw = x.shape
    p = dh * dw
    x3 = x.reshape(b, c, p)
    dct2 = dct_weight.reshape(c, p)
    k2d = k_tensor.reshape(b, 1)

    bb = 8
    grid = (b // bb,)

    s = pl.pallas_call(
        _pool_body,
        grid=grid,
        in_specs=[
            pl.BlockSpec((bb, c, p), lambda i: (i, 0, 0)),
            pl.BlockSpec((c, p), lambda i: (0, 0)),
        ],
        out_specs=pl.BlockSpec((bb, c), lambda i: (i, 0)),
        out_shape=jax.ShapeDtypeStruct((b, c), jnp.float32),
    )(x3, dct2)

    f = jax.ShapeDtypeStruct((b, c), jnp.float32)
    bounded, raw, mask, gate = pl.pallas_call(
        _gate_body,
        out_shape=(f, f, f, f),
    )(s, W1, W2, k2d)

    out3 = pl.pallas_call(
        _scale_body,
        grid=grid,
        in_specs=[
            pl.BlockSpec((bb, c, p), lambda i: (i, 0, 0)),
            pl.BlockSpec((bb, c), lambda i: (i, 0)),
        ],
        out_specs=pl.BlockSpec((bb, c, p), lambda i: (i, 0, 0)),
        out_shape=jax.ShapeDtypeStruct((b, c, p), jnp.float32),
    )(x3, gate)

    return (out3.reshape(b, c, dh, dw), bounded, raw, mask, s)


# speedup vs baseline: 1.7365x; 1.0013x over previous
import jax
import jax.numpy as jnp
from jax.experimental import pallas as pl
from jax.experimental.pallas import tpu as pltpu

def _pool_body(x_ref, d_ref, s_ref):
    s_ref[...] = jnp.sum(x_ref[...] * d_ref[...][None], axis=-1)


def _gate_body(s_ref, w1_ref, w2_ref, k_ref, b_ref, raw_ref, m_ref, g_ref):
    s = s_ref[...]
    h = jnp.maximum(jnp.dot(s, w1_ref[...].T, preferred_element_type=jnp.float32), 0.0)
    raw = jnp.dot(h, w2_ref[...].T, preferred_element_type=jnp.float32)
    raw_ref[...] = raw
    bounded = jnp.tanh(raw)
    b_ref[...] = bounded

    c = raw.shape[1]
    bits = jax.lax.bitcast_convert_type(raw + 0.0, jnp.int32)
    key = bits ^ ((bits >> 31) & jnp.int32(0x7FFFFFFF))
    k = k_ref[...]

    cnt0 = jnp.sum((key >= 0).astype(jnp.int32), axis=1, keepdims=True)
    base = jnp.where(cnt0 >= k, jnp.int32(0), jnp.int32(-2147483648))
    for bit in range(30, -1, -1):
        cand = base + jnp.int32(1 << bit)
        cnt = jnp.sum((key >= cand).astype(jnp.int32), axis=1, keepdims=True)
        base = jnp.where(cnt >= k, cand, base)

    gt = key > base
    eq = key == base
    cnt_gt = jnp.sum(gt.astype(jnp.int32), axis=1, keepdims=True)
    rows = jax.lax.broadcasted_iota(jnp.int32, (c, c), 0)
    cols = jax.lax.broadcasted_iota(jnp.int32, (c, c), 1)
    tri = (rows < cols).astype(jnp.float32)
    eq_prefix = jnp.dot(eq.astype(jnp.float32), tri,
                        preferred_element_type=jnp.float32)
    n_eq = (k - cnt_gt).astype(jnp.float32)
    mask = jnp.where(gt | (eq & (eq_prefix < n_eq)), 1.0, 0.0)
    m_ref[...] = mask
    g_ref[...] = bounded + (mask - bounded)


def _scale_body(x_ref, g_ref, o_ref):
    o_ref[...] = x_ref[...] * g_ref[...][:, :, None]


def kernel(x, dct_weight, W1, W2, k_tensor):
    b, c, dh, dw = x.shape
    p = dh * dw
    x3 = x.reshape(b, c, p)
    dct2 = dct_weight.reshape(c, p)
    k2d = k_tensor.reshape(b, 1)

    bb = 8
    grid = (b // bb,)

    s = pl.pallas_call(
        _pool_body,
        grid=grid,
        in_specs=[
            pl.BlockSpec((bb, c, p), lambda i: (i, 0, 0)),
            pl.BlockSpec((c, p), lambda i: (0, 0)),
        ],
        out_specs=pl.BlockSpec((bb, c), lambda i: (i, 0)),
        out_shape=jax.ShapeDtypeStruct((b, c), jnp.float32),
        compiler_params=pltpu.CompilerParams(
            dimension_semantics=("parallel",)),
    )(x3, dct2)

    f = jax.ShapeDtypeStruct((b, c), jnp.float32)
    bounded, raw, mask, gate = pl.pallas_call(
        _gate_body,
        out_shape=(f, f, f, f),
    )(s, W1, W2, k2d)

    out3 = pl.pallas_call(
        _scale_body,
        grid=grid,
        in_specs=[
            pl.BlockSpec((bb, c, p), lambda i: (i, 0, 0)),
            pl.BlockSpec((bb, c), lambda i: (i, 0)),
        ],
        out_specs=pl.BlockSpec((bb, c, p), lambda i: (i, 0, 0)),
        out_shape=jax.ShapeDtypeStruct((b, c, p), jnp.float32),
        compiler_params=pltpu.CompilerParams(
            dimension_semantics=("parallel",)),
    )(x3, gate)

    return (out3.reshape(b, c, dh, dw), bounded, raw, mask, s)
